# baseline (speedup 1.0000x reference)
import functools

import jax
import jax.numpy as jnp
from jax.experimental import pallas as pl


NUM_CODES = 1024
CODE_DIM = 256
COMMITMENT_COST = 0.25
ROWS = 512
KCHUNK = 256


def _vq_body(z_ref, zsq_ref, cb_ref, csq_ref, zq_ref, idx_ref, loss_ref):
    z = z_ref[...]
    z_sq = zsq_ref[...]
    k_total = cb_ref.shape[0]

    best_val = None
    best_idx = None
    for c in range(0, k_total, KCHUNK):
        cb_c = cb_ref[pl.ds(c, KCHUNK), :]
        c_sq = csq_ref[:, pl.ds(c, KCHUNK)]
        dot = jax.lax.dot_general(
            z, cb_c, (((1,), (1,)), ((), ())),
            preferred_element_type=jnp.float32)
        dist = z_sq - 2 * dot + c_sq
        mval = jnp.min(dist, axis=-1, keepdims=True)
        iota_k = c + jax.lax.broadcasted_iota(jnp.int32, dist.shape, 1)
        cidx = jnp.min(jnp.where(dist == mval, iota_k, k_total),
                       axis=-1, keepdims=True)
        if best_val is None:
            best_val, best_idx = mval, cidx
        else:
            take_new = mval < best_val
            best_idx = jnp.where(take_new, cidx, best_idx)
            best_val = jnp.where(take_new, mval, best_val)

    idx = best_idx[:, 0].astype(jnp.int32)
    onehot = (jax.lax.broadcasted_iota(jnp.int32, (z.shape[0], k_total), 1)
              == best_idx).astype(jnp.float32)
    z_q = jax.lax.dot_general(
        onehot, cb_ref[...], (((1,), (0,)), ((), ())),
        preferred_element_type=jnp.float32)
    zq_ref[...] = z + (z_q - z)
    idx_ref[...] = idx[:, None]
    diff = z_q - z
    part = jnp.sum(diff * diff).reshape(1, 1)

    @pl.when(pl.program_id(0) == 0)
    def _init():
        loss_ref[...] = part

    @pl.when(pl.program_id(0) != 0)
    def _acc():
        loss_ref[...] += part


@functools.partial(jax.jit, static_argnames=())
def kernel(z_e, codebook):
    B, N, D = z_e.shape
    K = codebook.shape[0]
    flat = z_e.reshape(B * N, D)
    nblk = (B * N) // ROWS
    z_sq = jnp.sum(z_e ** 2, axis=-1, keepdims=True).reshape(B * N, 1)
    c_sq = jnp.sum(codebook ** 2, axis=-1).reshape(1, K)

    zq_st, idx, loss_sum = pl.pallas_call(
        _vq_body,
        grid=(nblk,),
        in_specs=[
            pl.BlockSpec((ROWS, D), lambda i: (i, 0)),
            pl.BlockSpec((ROWS, 1), lambda i: (i, 0)),
            pl.BlockSpec((K, D), lambda i: (0, 0)),
            pl.BlockSpec((1, K), lambda i: (0, 0)),
        ],
        out_specs=[
            pl.BlockSpec((ROWS, D), lambda i: (i, 0)),
            pl.BlockSpec((ROWS, 1), lambda i: (i, 0)),
            pl.BlockSpec((1, 1), lambda i: (0, 0)),
        ],
        out_shape=[
            jax.ShapeDtypeStruct((B * N, D), jnp.float32),
            jax.ShapeDtypeStruct((B * N, 1), jnp.int32),
            jax.ShapeDtypeStruct((1, 1), jnp.float32),
        ],
    )(flat, z_sq, codebook, c_sq)

    mean_loss = loss_sum[0, 0] / (B * N * D)
    vq_loss = mean_loss + COMMITMENT_COST * mean_loss
    return (zq_st.reshape(B, N, D), idx.reshape(B, N), vq_loss)


# speedup vs baseline: 1.1423x; 1.1423x over previous
import functools

import jax
import jax.numpy as jnp
from jax.experimental import pallas as pl


NUM_CODES = 1024
CODE_DIM = 256
COMMITMENT_COST = 0.25
ROWS = 1024
KCHUNK = 1024


def _vq_body(z_ref, zsq_ref, cb_ref, csq_ref, zq_ref, idx_ref, loss_ref):
    z = z_ref[...]
    z_sq = zsq_ref[...]
    k_total = cb_ref.shape[0]

    best_val = None
    best_idx = None
    for c in range(0, k_total, KCHUNK):
        cb_c = cb_ref[pl.ds(c, KCHUNK), :]
        c_sq = csq_ref[:, pl.ds(c, KCHUNK)]
        dot = jax.lax.dot_general(
            z, cb_c, (((1,), (1,)), ((), ())),
            preferred_element_type=jnp.float32)
        dist = z_sq - 2 * dot + c_sq
        mval = jnp.min(dist, axis=-1, keepdims=True)
        iota_k = c + jax.lax.broadcasted_iota(jnp.int32, dist.shape, 1)
        cidx = jnp.min(jnp.where(dist == mval, iota_k, k_total),
                       axis=-1, keepdims=True)
        if best_val is None:
            best_val, best_idx = mval, cidx
        else:
            take_new = mval < best_val
            best_idx = jnp.where(take_new, cidx, best_idx)
            best_val = jnp.where(take_new, mval, best_val)

    idx = best_idx[:, 0].astype(jnp.int32)
    onehot = (jax.lax.broadcasted_iota(jnp.int32, (z.shape[0], k_total), 1)
              == best_idx).astype(jnp.float32)
    z_q = jax.lax.dot_general(
        onehot, cb_ref[...], (((1,), (0,)), ((), ())),
        preferred_element_type=jnp.float32)
    zq_ref[...] = z + (z_q - z)
    idx_ref[...] = idx[:, None]
    diff = z_q - z
    part = jnp.sum(diff * diff).reshape(1, 1)

    @pl.when(pl.program_id(0) == 0)
    def _init():
        loss_ref[...] = part

    @pl.when(pl.program_id(0) != 0)
    def _acc():
        loss_ref[...] += part


@functools.partial(jax.jit, static_argnames=())
def kernel(z_e, codebook):
    B, N, D = z_e.shape
    K = codebook.shape[0]
    flat = z_e.reshape(B * N, D)
    nblk = (B * N) // ROWS
    z_sq = jnp.sum(z_e ** 2, axis=-1, keepdims=True).reshape(B * N, 1)
    c_sq = jnp.sum(codebook ** 2, axis=-1).reshape(1, K)

    zq_st, idx, loss_sum = pl.pallas_call(
        _vq_body,
        grid=(nblk,),
        in_specs=[
            pl.BlockSpec((ROWS, D), lambda i: (i, 0)),
            pl.BlockSpec((ROWS, 1), lambda i: (i, 0)),
            pl.BlockSpec((K, D), lambda i: (0, 0)),
            pl.BlockSpec((1, K), lambda i: (0, 0)),
        ],
        out_specs=[
            pl.BlockSpec((ROWS, D), lambda i: (i, 0)),
            pl.BlockSpec((ROWS, 1), lambda i: (i, 0)),
            pl.BlockSpec((1, 1), lambda i: (0, 0)),
        ],
        out_shape=[
            jax.ShapeDtypeStruct((B * N, D), jnp.float32),
            jax.ShapeDtypeStruct((B * N, 1), jnp.int32),
            jax.ShapeDtypeStruct((1, 1), jnp.float32),
        ],
    )(flat, z_sq, codebook, c_sq)

    mean_loss = loss_sum[0, 0] / (B * N * D)
    vq_loss = mean_loss + COMMITMENT_COST * mean_loss
    return (zq_st.reshape(B, N, D), idx.reshape(B, N), vq_loss)


# speedup vs baseline: 1.2054x; 1.0553x over previous
import functools

import jax
import jax.numpy as jnp
from jax.experimental import pallas as pl


NUM_CODES = 1024
CODE_DIM = 256
COMMITMENT_COST = 0.25
ROWS = 2048
KCHUNK = 1024


def _vq_body(z_ref, zsq_ref, cb_ref, csq_ref, zq_ref, idx_ref, loss_ref):
    z = z_ref[...]
    z_sq = zsq_ref[...]
    k_total = cb_ref.shape[0]

    best_val = None
    best_idx = None
    for c in range(0, k_total, KCHUNK):
        cb_c = cb_ref[pl.ds(c, KCHUNK), :]
        c_sq = csq_ref[:, pl.ds(c, KCHUNK)]
        dot = jax.lax.dot_general(
            z, cb_c, (((1,), (1,)), ((), ())),
            preferred_element_type=jnp.float32)
        dist = z_sq - 2 * dot + c_sq
        mval = jnp.min(dist, axis=-1, keepdims=True)
        iota_k = c + jax.lax.broadcasted_iota(jnp.int32, dist.shape, 1)
        cidx = jnp.min(jnp.where(dist == mval, iota_k, k_total),
                       axis=-1, keepdims=True)
        if best_val is None:
            best_val, best_idx = mval, cidx
        else:
            take_new = mval < best_val
            best_idx = jnp.where(take_new, cidx, best_idx)
            best_val = jnp.where(take_new, mval, best_val)

    idx = best_idx[:, 0].astype(jnp.int32)
    onehot = (jax.lax.broadcasted_iota(jnp.int32, (z.shape[0], k_total), 1)
              == best_idx).astype(jnp.float32)
    z_q = jax.lax.dot_general(
        onehot, cb_ref[...], (((1,), (0,)), ((), ())),
        preferred_element_type=jnp.float32)
    zq_ref[...] = z + (z_q - z)
    idx_ref[...] = idx[:, None]
    diff = z_q - z
    part = jnp.sum(diff * diff).reshape(1, 1)

    @pl.when(pl.program_id(0) == 0)
    def _init():
        loss_ref[...] = part

    @pl.when(pl.program_id(0) != 0)
    def _acc():
        loss_ref[...] += part


@functools.partial(jax.jit, static_argnames=())
def kernel(z_e, codebook):
    B, N, D = z_e.shape
    K = codebook.shape[0]
    flat = z_e.reshape(B * N, D)
    nblk = (B * N) // ROWS
    z_sq = jnp.sum(z_e ** 2, axis=-1, keepdims=True).reshape(B * N, 1)
    c_sq = jnp.sum(codebook ** 2, axis=-1).reshape(1, K)

    zq_st, idx, loss_sum = pl.pallas_call(
        _vq_body,
        grid=(nblk,),
        in_specs=[
            pl.BlockSpec((ROWS, D), lambda i: (i, 0)),
            pl.BlockSpec((ROWS, 1), lambda i: (i, 0)),
            pl.BlockSpec((K, D), lambda i: (0, 0)),
            pl.BlockSpec((1, K), lambda i: (0, 0)),
        ],
        out_specs=[
            pl.BlockSpec((ROWS, D), lambda i: (i, 0)),
            pl.BlockSpec((ROWS, 1), lambda i: (i, 0)),
            pl.BlockSpec((1, 1), lambda i: (0, 0)),
        ],
        out_shape=[
            jax.ShapeDtypeStruct((B * N, D), jnp.float32),
            jax.ShapeDtypeStruct((B * N, 1), jnp.int32),
            jax.ShapeDtypeStruct((1, 1), jnp.float32),
        ],
    )(flat, z_sq, codebook, c_sq)

    mean_loss = loss_sum[0, 0] / (B * N * D)
    vq_loss = mean_loss + COMMITMENT_COST * mean_loss
    return (zq_st.reshape(B, N, D), idx.reshape(B, N), vq_loss)
